# baseline (speedup 1.0000x reference)
import functools

import jax
import jax.numpy as jnp
from jax import lax
from jax.experimental import pallas as pl
from jax.experimental.pallas import tpu as pltpu
from jax.experimental.pallas import tpu_sc as plsc

_NUM_CORES = 2
_NUM_SUBCORES = 16
_NW = _NUM_CORES * _NUM_SUBCORES
_CHUNKS = (128, 128, 128, 64, 64)
_BUF = (0, 1, 2, 3, 3)
_NBUF = 4


@functools.lru_cache(maxsize=None)
def _make_gather(b0: int, b1: int, d: int):
    b = b0 * b1
    b_per_w = b // _NW
    assert b % _NW == 0 and _NW % b0 == 0
    assert sum(_CHUNKS) == b_per_w
    n_chunks = len(_CHUNKS)
    starts = [sum(_CHUNKS[:c]) for c in range(n_chunks)]
    w_per_row = _NW // b0
    row_bytes = d * 4

    mesh = plsc.VectorSubcoreMesh(core_axis_name="c", subcore_axis_name="s")
    scratch = [pltpu.VMEM((b_per_w,), jnp.int32)]
    scratch += [pltpu.VMEM((_CHUNKS[bi], d), jnp.float32)
                for bi in range(_NBUF)]
    scratch += [pltpu.SemaphoreType.DMA for _ in range(5)]

    @functools.partial(
        pl.kernel,
        mesh=mesh,
        out_type=jax.ShapeDtypeStruct((b, d), jnp.float32),
        scratch_types=scratch,
    )
    def gather_kernel(idx_hbm, table_hbm, out_hbm, idx_v,
                      buf0, buf1, buf2, buf3, isem0, isem1, gsem, ssem,
                      rsem):
        bufs = (buf0, buf1, buf2, buf3)
        wid = lax.axis_index("s") * _NUM_CORES + lax.axis_index("c")
        base = wid * b_per_w
        row = wid // w_per_row
        col = (wid % w_per_row) * b_per_w
        head = _CHUNKS[0]

        def idx_copy(lo, n, sem):
            return pltpu.make_async_copy(
                idx_hbm.at[row, pl.ds(col + lo, n)],
                idx_v.at[pl.ds(lo, n)], sem)

        def gather_copy(c):
            return pltpu.make_async_copy(
                table_hbm.at[idx_v.at[pl.ds(starts[c], _CHUNKS[c])]],
                bufs[_BUF[c]], gsem)

        def scatter_copy(c, sem):
            return pltpu.make_async_copy(
                bufs[_BUF[c]],
                out_hbm.at[pl.ds(base + starts[c], _CHUNKS[c])], sem)

        idx_copy(0, head, isem0).start()
        idx_copy(head, b_per_w - head, isem1).start()
        idx_copy(0, head, isem0).wait()
        gather_copy(0).start()
        idx_copy(head, b_per_w - head, isem1).wait()
        for c in range(1, 4):
            gather_copy(c).start()
        for c in range(4):
            gather_copy(c).wait()

        scatter_copy(3, rsem).start()
        for c in range(3):
            scatter_copy(c, ssem).start()
        scatter_copy(3, rsem).wait()

        gather_copy(4).start()
        gather_copy(4).wait()
        scatter_copy(4, ssem).start()

        for c in range(3):
            scatter_copy(c, ssem).wait()
        scatter_copy(4, ssem).wait()

    return gather_kernel


def kernel(embed_id, weight):
    b0, b1 = embed_id.shape
    d = weight.shape[1]
    idx = jnp.asarray(embed_id, jnp.int32)
    out = _make_gather(b0, b1, d)(idx, weight)
    return out.reshape(b0, b1, d)


# speedup vs baseline: 1.0068x; 1.0068x over previous
import functools

import jax
import jax.numpy as jnp
from jax import lax
from jax.experimental import pallas as pl
from jax.experimental.pallas import tpu as pltpu
from jax.experimental.pallas import tpu_sc as plsc

_NUM_CORES = 2
_NUM_SUBCORES = 16
_NW = _NUM_CORES * _NUM_SUBCORES
_CHUNKS = (128, 128, 128, 64, 64)
_BUF = (0, 1, 2, 3, 3)
_NBUF = 4


@functools.lru_cache(maxsize=None)
def _make_gather(b0: int, b1: int, d: int):
    b = b0 * b1
    b_per_w = b // _NW
    assert b % _NW == 0 and _NW % b0 == 0
    assert sum(_CHUNKS) == b_per_w
    n_chunks = len(_CHUNKS)
    starts = [sum(_CHUNKS[:c]) for c in range(n_chunks)]
    w_per_row = _NW // b0

    mesh = plsc.VectorSubcoreMesh(core_axis_name="c", subcore_axis_name="s")
    scratch = [pltpu.VMEM((b_per_w,), jnp.int32)]
    scratch += [pltpu.VMEM((_CHUNKS[bi], d), jnp.float32)
                for bi in range(_NBUF)]
    scratch += [pltpu.SemaphoreType.DMA for _ in range(5)]

    @functools.partial(
        pl.kernel,
        mesh=mesh,
        out_type=jax.ShapeDtypeStruct((b, d), jnp.float32),
        scratch_types=scratch,
    )
    def gather_kernel(idx_hbm, table_hbm, out_hbm, idx_v,
                      buf0, buf1, buf2, buf3, isem0, isem1, gsem, ssem,
                      rsem):
        bufs = (buf0, buf1, buf2, buf3)
        wid = lax.axis_index("s") * _NUM_CORES + lax.axis_index("c")
        base = wid * b_per_w
        row = wid // w_per_row
        col = (wid % w_per_row) * b_per_w
        head = _CHUNKS[0]

        def idx_copy(lo, n, sem):
            return pltpu.make_async_copy(
                idx_hbm.at[row, pl.ds(col + lo, n)],
                idx_v.at[pl.ds(lo, n)], sem)

        def gather_copy(c):
            return pltpu.make_async_copy(
                table_hbm.at[idx_v.at[pl.ds(starts[c], _CHUNKS[c])]],
                bufs[_BUF[c]], gsem)

        def scatter_copy(c, sem):
            return pltpu.make_async_copy(
                bufs[_BUF[c]],
                out_hbm.at[pl.ds(base + starts[c], _CHUNKS[c])], sem)

        idx_copy(0, head, isem0).start()
        idx_copy(head, b_per_w - head, isem1).start()
        idx_copy(0, head, isem0).wait()
        gather_copy(0).start()
        idx_copy(head, b_per_w - head, isem1).wait()
        for c in range(1, 4):
            gather_copy(c).start()
        for c in range(4):
            gather_copy(c).wait()

        scatter_copy(3, rsem).start()
        for c in range(3):
            scatter_copy(c, ssem).start()
        scatter_copy(3, rsem).wait()

        gather_copy(4).start()
        gather_copy(4).wait()
        scatter_copy(4, ssem).start()

        for c in range(3):
            scatter_copy(c, ssem).wait()
        scatter_copy(4, ssem).wait()

    return gather_kernel


def kernel(embed_id, weight):
    b0, b1 = embed_id.shape
    d = weight.shape[1]
    idx = jnp.asarray(embed_id, jnp.int32)
    out = _make_gather(b0, b1, d)(idx, weight)
    return out.reshape(b0, b1, d)
